# baseline (speedup 1.0000x reference)
import functools

import jax
import jax.numpy as jnp
from jax import lax
from jax.experimental import pallas as pl
from jax.experimental.pallas import tpu as pltpu
from jax.experimental.pallas import tpu_sc as plsc

_NC = 2
_NS = 16
_NW = _NC * _NS
_L = 16

_CH = 32
_EPS = 1e-5


_GDN = lax.GatherDimensionNumbers(
    offset_dims=(), collapsed_slice_dims=(0,), start_index_map=(0,))


def _rotate(v, shift):
    idx = (lax.iota(jnp.int32, _L) + shift) & (_L - 1)
    return lax.gather(v, idx[:, None], dimension_numbers=_GDN,
                      slice_sizes=(1,),
                      mode=lax.GatherScatterMode.PROMISE_IN_BOUNDS)


def _allreduce_sum(v):
    for shift in (1, 2, 4, 8):
        v = v + _rotate(v, shift)
    return v


def _rsqrt_v(x):
    i = lax.bitcast_convert_type(x, jnp.int32)
    i = jnp.int32(0x5F3759DF) - lax.shift_right_logical(i, 1)
    y = lax.bitcast_convert_type(i, jnp.float32)
    for _ in range(3):
        y = y * (1.5 - 0.5 * x * y * y)
    return y


def _make_sc_kernel(N, E, S):
    tpw = N // _NW
    nchunk = tpw // _CH
    ne = E // _L
    inv_e = 1.0 / E
    mesh = plsc.VectorSubcoreMesh(core_axis_name="c", subcore_axis_name="s")

    @functools.partial(
        pl.kernel,
        mesh=mesh,
        out_type=jax.ShapeDtypeStruct((N, E), jnp.float32),
        scratch_types=[
            pltpu.VMEM((_CH,), jnp.int32),
            pltpu.VMEM((_CH,), jnp.int32),
            pltpu.VMEM((_CH, E), jnp.float32),
            pltpu.VMEM((_CH, E), jnp.float32),
            pltpu.VMEM((_CH, E), jnp.float32),
            pltpu.VMEM((E,), jnp.float32),
            pltpu.VMEM((E,), jnp.float32),
            pltpu.SemaphoreType.DMA,
            pltpu.SemaphoreType.DMA,
        ],
    )
    def emb_kernel(seq_hbm, seg_hbm, tok_tab, pos_tab, seg_tab, gamma_hbm,
                   beta_hbm, out_hbm, idx_v, segidx_v, buf_tok, buf_pos,
                   buf_seg, gamma_v, beta_v, sem_a, sem_b):
        wid = lax.axis_index("s") * _NC + lax.axis_index("c")

        pltpu.sync_copy(gamma_hbm, gamma_v)
        pltpu.sync_copy(beta_hbm, beta_v)

        def token_body(t, carry):
            acc = jnp.zeros((_L,), jnp.float32)
            acc2 = jnp.zeros((_L,), jnp.float32)
            xs = []
            for k in range(ne):
                sl = pl.ds(_L * k, _L)
                x = buf_tok[t, sl] + buf_pos[t, sl] + buf_seg[t, sl]
                xs.append(x)
                acc = acc + x
                acc2 = acc2 + x * x
            mean_v = _allreduce_sum(acc) * inv_e
            var_v = _allreduce_sum(acc2) * inv_e - mean_v * mean_v
            rs_v = _rsqrt_v(var_v + _EPS)
            for k in range(ne):
                sl = pl.ds(_L * k, _L)
                buf_tok[t, sl] = ((xs[k] - mean_v) * rs_v * gamma_v[sl]
                                  + beta_v[sl])
            return carry

        def chunk_body(c, carry):
            base = wid * tpw + c * _CH
            pltpu.sync_copy(seq_hbm.at[pl.ds(base, _CH)], idx_v)
            pltpu.sync_copy(seg_hbm.at[pl.ds(base, _CH)], segidx_v)
            pltpu.sync_copy(pos_tab.at[pl.ds(c * _CH, _CH)], buf_pos)
            cp_tok = pltpu.async_copy(tok_tab.at[idx_v], buf_tok, sem_a)
            cp_seg = pltpu.async_copy(seg_tab.at[segidx_v], buf_seg, sem_b)
            cp_tok.wait()
            cp_seg.wait()
            lax.fori_loop(0, _CH, token_body, 0)
            pltpu.sync_copy(buf_tok, out_hbm.at[pl.ds(base, _CH)])
            return carry

        lax.fori_loop(0, nchunk, chunk_body, 0)

    return emb_kernel


def kernel(sequence, segment_label, token_table, pos_table, seg_table,
           ln_gamma, ln_beta):
    B, S = sequence.shape
    E = token_table.shape[1]
    N = B * S
    seq_flat = sequence.reshape(N).astype(jnp.int32)
    seg_flat = segment_label.reshape(N).astype(jnp.int32)
    emb = _make_sc_kernel(N, E, S)
    out = emb(seq_flat, seg_flat, token_table, pos_table, seg_table,
              jnp.asarray(ln_gamma, jnp.float32),
              jnp.asarray(ln_beta, jnp.float32))
    return out.reshape(B, S, E)


# speedup vs baseline: 1.0024x; 1.0024x over previous
import functools

import jax
import jax.numpy as jnp
from jax import lax
from jax.experimental import pallas as pl
from jax.experimental.pallas import tpu as pltpu
from jax.experimental.pallas import tpu_sc as plsc

_NC = 2
_NS = 16
_NW = _NC * _NS
_L = 16

_CH = 32
_EPS = 1e-5


_GDN = lax.GatherDimensionNumbers(
    offset_dims=(), collapsed_slice_dims=(0,), start_index_map=(0,))


def _rotate(v, shift):
    idx = (lax.iota(jnp.int32, _L) + shift) & (_L - 1)
    return lax.gather(v, idx[:, None], dimension_numbers=_GDN,
                      slice_sizes=(1,),
                      mode=lax.GatherScatterMode.PROMISE_IN_BOUNDS)


def _allreduce_sum(v):
    for shift in (1, 2, 4, 8):
        v = v + _rotate(v, shift)
    return v


def _rsqrt_v(x):
    i = lax.bitcast_convert_type(x, jnp.int32)
    i = jnp.int32(0x5F3759DF) - lax.shift_right_logical(i, 1)
    y = lax.bitcast_convert_type(i, jnp.float32)
    for _ in range(3):
        y = y * (1.5 - 0.5 * x * y * y)
    return y


def _make_sc_kernel(N, E, S):
    tpw = N // _NW
    nchunk = tpw // _CH
    ne = E // _L
    inv_e = 1.0 / E
    mesh = plsc.VectorSubcoreMesh(core_axis_name="c", subcore_axis_name="s")

    @functools.partial(
        pl.kernel,
        mesh=mesh,
        out_type=jax.ShapeDtypeStruct((N, E), jnp.float32),
        scratch_types=[
            pltpu.VMEM((_CH,), jnp.int32),
            pltpu.VMEM((_CH,), jnp.int32),
            pltpu.VMEM((_CH, E), jnp.float32),
            pltpu.VMEM((_CH, E), jnp.float32),
            pltpu.VMEM((_CH, E), jnp.float32),
            pltpu.VMEM((E,), jnp.float32),
            pltpu.VMEM((E,), jnp.float32),
            pltpu.SemaphoreType.DMA,
            pltpu.SemaphoreType.DMA,
        ],
    )
    def emb_kernel(seq_hbm, seg_hbm, tok_tab, pos_tab, seg_tab, gamma_hbm,
                   beta_hbm, out_hbm, idx_v, segidx_v, buf_tok, buf_pos,
                   buf_seg, gamma_v, beta_v, sem_a, sem_b):
        wid = lax.axis_index("s") * _NC + lax.axis_index("c")

        pltpu.sync_copy(gamma_hbm, gamma_v)
        pltpu.sync_copy(beta_hbm, beta_v)

        def token_body(t, carry):
            acc = jnp.zeros((_L,), jnp.float32)
            acc2 = jnp.zeros((_L,), jnp.float32)
            for k in range(ne):
                sl = pl.ds(_L * k, _L)
                x = buf_tok[t, sl] + buf_pos[t, sl] + buf_seg[t, sl]
                buf_tok[t, sl] = x
                acc = acc + x
                acc2 = acc2 + x * x
            mean_v = _allreduce_sum(acc) * inv_e
            var_v = _allreduce_sum(acc2) * inv_e - mean_v * mean_v
            rs_v = _rsqrt_v(var_v + _EPS)
            for k in range(ne):
                sl = pl.ds(_L * k, _L)
                buf_tok[t, sl] = ((buf_tok[t, sl] - mean_v) * rs_v
                                  * gamma_v[sl] + beta_v[sl])
            return carry

        def chunk_body(c, carry):
            base = wid * tpw + c * _CH
            pltpu.sync_copy(seq_hbm.at[pl.ds(base, _CH)], idx_v)
            pltpu.sync_copy(seg_hbm.at[pl.ds(base, _CH)], segidx_v)
            pltpu.sync_copy(pos_tab.at[pl.ds(c * _CH, _CH)], buf_pos)
            cp_tok = pltpu.async_copy(tok_tab.at[idx_v], buf_tok, sem_a)
            cp_seg = pltpu.async_copy(seg_tab.at[segidx_v], buf_seg, sem_b)
            cp_tok.wait()
            cp_seg.wait()
            lax.fori_loop(0, _CH, token_body, 0)
            pltpu.sync_copy(buf_tok, out_hbm.at[pl.ds(base, _CH)])
            return carry

        lax.fori_loop(0, nchunk, chunk_body, 0)

    return emb_kernel


def kernel(sequence, segment_label, token_table, pos_table, seg_table,
           ln_gamma, ln_beta):
    B, S = sequence.shape
    E = token_table.shape[1]
    N = B * S
    seq_flat = sequence.reshape(N).astype(jnp.int32)
    seg_flat = segment_label.reshape(N).astype(jnp.int32)
    emb = _make_sc_kernel(N, E, S)
    out = emb(seq_flat, seg_flat, token_table, pos_table, seg_table,
              jnp.asarray(ln_gamma, jnp.float32),
              jnp.asarray(ln_beta, jnp.float32))
    return out.reshape(B, S, E)
